# baseline (speedup 1.0000x reference)
import functools

import jax
import jax.numpy as jnp
from jax.experimental import pallas as pl
from jax.experimental.pallas import tpu as pltpu
from jax.experimental.pallas import tpu_sc as plsc

_B, _W, _D = 1024, 200, 64
_HIST, _FUT = 150, 50
_V1 = 1000
_HC = 7
_FC = 3
_HIST_CH = [7, None, 5, 6, 2, 3, 4]
_FUT_CH = [None, 5, 6]
_BB = 8

_NC, _NS = 2, 16
_NW = _NC * _NS


def _tc_body(x_ref, m_ref, bias_ref, mf_ref, biasf_ref, e1_ref, hist_ref, fut_ref):
    for b in range(_BB):
        xb = x_ref[b]
        ih = xb[:, 1].astype(jnp.int16)
        oh = (ih[:, None] == jax.lax.broadcasted_iota(jnp.int16, (_W, _V1), 1)).astype(jnp.bfloat16)
        e1 = jnp.dot(oh, e1_ref[...], preferred_element_type=jnp.float32)
        dh = jnp.dot(xb[:_HIST], m_ref[...], preferred_element_type=jnp.float32,
                     precision=jax.lax.Precision.HIGHEST) + bias_ref[...]
        df = jnp.dot(xb[_HIST:], mf_ref[...], preferred_element_type=jnp.float32,
                     precision=jax.lax.Precision.HIGHEST) + biasf_ref[...]
        hist_ref[b] = dh
        hist_ref[b, :, _D:2 * _D] = e1[:_HIST]
        fut_ref[b] = df
        fut_ref[b, :, 0:_D] = e1[_HIST:]


def _dense_outputs(inputs, e1_bf, m, bias, mf, biasf):
    return pl.pallas_call(
        _tc_body,
        grid=(_B // _BB,),
        in_specs=[
            pl.BlockSpec((_BB, _W, 8), lambda i: (i, 0, 0)),
            pl.BlockSpec((8, _HC * _D), lambda i: (0, 0)),
            pl.BlockSpec((1, _HC * _D), lambda i: (0, 0)),
            pl.BlockSpec((8, _FC * _D), lambda i: (0, 0)),
            pl.BlockSpec((1, _FC * _D), lambda i: (0, 0)),
            pl.BlockSpec((_V1, _D), lambda i: (0, 0)),
        ],
        out_specs=[
            pl.BlockSpec((_BB, _HIST, _HC * _D), lambda i: (i, 0, 0)),
            pl.BlockSpec((_BB, _FUT, _FC * _D), lambda i: (i, 0, 0)),
        ],
        out_shape=[
            jax.ShapeDtypeStruct((_B, _HIST, _HC * _D), jnp.float32),
            jax.ShapeDtypeStruct((_B, _FUT, _FC * _D), jnp.float32),
        ],
    )(inputs, m, bias, mf, biasf, e1_bf)


def _static_gather(idx0, E0):
    bpw = _B // _NW
    mesh = plsc.VectorSubcoreMesh(core_axis_name="c", subcore_axis_name="s")

    @functools.partial(
        pl.kernel,
        mesh=mesh,
        out_type=jax.ShapeDtypeStruct((_B, _D), jnp.float32),
        compiler_params=pltpu.CompilerParams(use_tc_tiling_on_sc=False),
        scratch_types=[
            pltpu.VMEM((bpw,), jnp.int32),
            pltpu.VMEM((bpw, _D), jnp.float32),
            pltpu.SemaphoreType.DMA,
        ],
    )
    def k(idx_hbm, table_hbm, out_hbm, idx_v, rows_v, sem):
        wid = jax.lax.axis_index("s") * _NC + jax.lax.axis_index("c")
        base = wid * bpw
        pltpu.sync_copy(idx_hbm.at[pl.ds(base, bpw)], idx_v)
        pltpu.async_copy(table_hbm.at[idx_v], rows_v, sem).wait()
        pltpu.sync_copy(rows_v, out_hbm.at[pl.ds(base, bpw)])

    return k(idx0, E0)


def kernel(inputs, E0, E1, W2, b2, W3, b3, W4, b4, W5, b5, W6, b6, W7, b7):
    ws = {2: (W2, b2), 3: (W3, b3), 4: (W4, b4), 5: (W5, b5), 6: (W6, b6), 7: (W7, b7)}

    def proj(chans):
        mcols, bcols = [], []
        for v in chans:
            if v is None:
                mcols.append(jnp.zeros((8, _D), jnp.float32))
                bcols.append(jnp.zeros((_D,), jnp.float32))
            else:
                wv, bv = ws[v]
                mcols.append(jnp.zeros((8, _D), jnp.float32).at[v].set(wv[0]))
                bcols.append(bv)
        return jnp.concatenate(mcols, axis=1), jnp.concatenate(bcols)[None, :]

    m, bias = proj(_HIST_CH)
    mf, biasf = proj(_FUT_CH)
    e1_bf = E1.astype(jnp.bfloat16)

    hist, fut = _dense_outputs(inputs, e1_bf, m, bias, mf, biasf)
    idx0 = inputs[:, 0, 0].astype(jnp.int32)
    static = _static_gather(idx0, E0)

    return (
        static.reshape(_B, 1, _D),
        hist.reshape(_B, _HIST, _HC, _D),
        fut.reshape(_B, _FUT, _FC, _D),
    )


# speedup vs baseline: 1.0212x; 1.0212x over previous
import functools

import jax
import jax.numpy as jnp
from jax.experimental import pallas as pl
from jax.experimental.pallas import tpu as pltpu
from jax.experimental.pallas import tpu_sc as plsc

_B, _W, _D = 1024, 200, 64
_HIST, _FUT = 150, 50
_V1 = 1000
_HC = 7
_FC = 3
_HIST_CH = [7, None, 5, 6, 2, 3, 4]
_FUT_CH = [None, 5, 6]
_BB = 8

_NC, _NS = 2, 16
_NW = _NC * _NS
_CHE = 128


def _tc_body(x_ref, stage_ref, m_ref, bias_ref, mf_ref, biasf_ref, hist_ref, fut_ref):
    for b in range(_BB):
        xb = x_ref[b]
        e1 = stage_ref[b]
        dh = jnp.dot(xb[:_HIST], m_ref[...], preferred_element_type=jnp.float32,
                     precision=jax.lax.Precision.HIGHEST) + bias_ref[...]
        df = jnp.dot(xb[_HIST:], mf_ref[...], preferred_element_type=jnp.float32,
                     precision=jax.lax.Precision.HIGHEST) + biasf_ref[...]
        hist_ref[b] = dh
        hist_ref[b, :, _D:2 * _D] = e1[:_HIST]
        fut_ref[b] = df
        fut_ref[b, :, 0:_D] = e1[_HIST:]


def _dense_outputs(inputs, stage, m, bias, mf, biasf):
    return pl.pallas_call(
        _tc_body,
        grid=(_B // _BB,),
        in_specs=[
            pl.BlockSpec((_BB, _W, 8), lambda i: (i, 0, 0)),
            pl.BlockSpec((_BB, _W, _D), lambda i: (i, 0, 0)),
            pl.BlockSpec((8, _HC * _D), lambda i: (0, 0)),
            pl.BlockSpec((1, _HC * _D), lambda i: (0, 0)),
            pl.BlockSpec((8, _FC * _D), lambda i: (0, 0)),
            pl.BlockSpec((1, _FC * _D), lambda i: (0, 0)),
        ],
        out_specs=[
            pl.BlockSpec((_BB, _HIST, _HC * _D), lambda i: (i, 0, 0)),
            pl.BlockSpec((_BB, _FUT, _FC * _D), lambda i: (i, 0, 0)),
        ],
        out_shape=[
            jax.ShapeDtypeStruct((_B, _HIST, _HC * _D), jnp.float32),
            jax.ShapeDtypeStruct((_B, _FUT, _FC * _D), jnp.float32),
        ],
    )(inputs, stage, m, bias, mf, biasf)


def _static_gather(idx0, E0):
    bpw = _B // _NW
    mesh = plsc.VectorSubcoreMesh(core_axis_name="c", subcore_axis_name="s")

    @functools.partial(
        pl.kernel,
        mesh=mesh,
        out_type=jax.ShapeDtypeStruct((_B, _D), jnp.float32),
        compiler_params=pltpu.CompilerParams(use_tc_tiling_on_sc=False),
        scratch_types=[
            pltpu.VMEM((bpw,), jnp.int32),
            pltpu.VMEM((bpw, _D), jnp.float32),
            pltpu.SemaphoreType.DMA,
        ],
    )
    def k(idx_hbm, table_hbm, out_hbm, idx_v, rows_v, sem):
        wid = jax.lax.axis_index("s") * _NC + jax.lax.axis_index("c")
        base = wid * bpw
        pltpu.sync_copy(idx_hbm.at[pl.ds(base, bpw)], idx_v)
        pltpu.async_copy(table_hbm.at[idx_v], rows_v, sem).wait()
        pltpu.sync_copy(rows_v, out_hbm.at[pl.ds(base, bpw)])

    return k(idx0, E0)


def _e1_stage(idx1, E1):
    n = _B * _W
    npw = n // _NW
    nch = npw // _CHE
    mesh = plsc.VectorSubcoreMesh(core_axis_name="c", subcore_axis_name="s")

    @functools.partial(
        pl.kernel,
        mesh=mesh,
        out_type=jax.ShapeDtypeStruct((n, _D), jnp.float32),
        compiler_params=pltpu.CompilerParams(use_tc_tiling_on_sc=False),
        scratch_types=[
            pltpu.VMEM((2, _CHE), jnp.int32),
            pltpu.VMEM((2, _CHE, _D), jnp.float32),
            pltpu.SemaphoreType.DMA,
            pltpu.SemaphoreType.DMA,
            pltpu.SemaphoreType.DMA,
        ],
    )
    def k(idx_hbm, table_hbm, out_hbm, idx_v, rows_v, gsem, wsem0, wsem1):
        wid = jax.lax.axis_index("s") * _NC + jax.lax.axis_index("c")
        base = wid * npw
        wsems = (wsem0, wsem1)

        def gather(ch, sl):
            row0 = base + ch * _CHE
            pltpu.sync_copy(idx_hbm.at[pl.ds(row0, _CHE)], idx_v.at[sl])
            return pltpu.async_copy(table_hbm.at[idx_v.at[sl]], rows_v.at[sl], gsem)

        g = gather(0, 0)
        pend = [None, None]
        for ch in range(nch):
            sl = ch % 2
            g.wait()
            nxt = 1 - sl
            if ch + 1 < nch:
                if pend[nxt] is not None:
                    pend[nxt].wait()
                    pend[nxt] = None
                g = gather(ch + 1, nxt)
            pend[sl] = pltpu.async_copy(
                rows_v.at[sl], out_hbm.at[pl.ds(base + ch * _CHE, _CHE)], wsems[sl])
        for sl in range(2):
            if pend[sl] is not None:
                pend[sl].wait()

    return k(idx1, E1)


def kernel(inputs, E0, E1, W2, b2, W3, b3, W4, b4, W5, b5, W6, b6, W7, b7):
    ws = {2: (W2, b2), 3: (W3, b3), 4: (W4, b4), 5: (W5, b5), 6: (W6, b6), 7: (W7, b7)}

    def proj(chans):
        mcols, bcols = [], []
        for v in chans:
            if v is None:
                mcols.append(jnp.zeros((8, _D), jnp.float32))
                bcols.append(jnp.zeros((_D,), jnp.float32))
            else:
                wv, bv = ws[v]
                mcols.append(jnp.zeros((8, _D), jnp.float32).at[v].set(wv[0]))
                bcols.append(bv)
        return jnp.concatenate(mcols, axis=1), jnp.concatenate(bcols)[None, :]

    m, bias = proj(_HIST_CH)
    mf, biasf = proj(_FUT_CH)

    idx1 = inputs[:, :, 1].astype(jnp.int32).reshape(-1)
    stage = _e1_stage(idx1, E1).reshape(_B, _W, _D)
    hist, fut = _dense_outputs(inputs, stage, m, bias, mf, biasf)

    idx0 = inputs[:, 0, 0].astype(jnp.int32)
    static = _static_gather(idx0, E0)

    return (
        static.reshape(_B, 1, _D),
        hist.reshape(_B, _HIST, _HC, _D),
        fut.reshape(_B, _FUT, _FC, _D),
    )


# speedup vs baseline: 1.0330x; 1.0116x over previous
import functools

import jax
import jax.numpy as jnp
from jax.experimental import pallas as pl
from jax.experimental.pallas import tpu as pltpu
from jax.experimental.pallas import tpu_sc as plsc

_B, _W, _D = 1024, 200, 64
_HIST, _FUT = 150, 50
_V1 = 1000
_HC = 7
_FC = 3
_HIST_CH = [7, None, 5, 6, 2, 3, 4]
_FUT_CH = [None, 5, 6]
_BB = 16

_NC, _NS = 2, 16
_NW = _NC * _NS
_CHE = 128


def _tc_body(x_ref, stage_ref, m_ref, bias_ref, mf_ref, biasf_ref, hist_ref, fut_ref):
    for b in range(_BB):
        xb = x_ref[b]
        e1 = stage_ref[b]
        dh = jnp.dot(xb[:_HIST], m_ref[...], preferred_element_type=jnp.float32,
                     precision=jax.lax.Precision.HIGHEST) + bias_ref[...]
        df = jnp.dot(xb[_HIST:], mf_ref[...], preferred_element_type=jnp.float32,
                     precision=jax.lax.Precision.HIGHEST) + biasf_ref[...]
        hist_ref[b] = dh
        hist_ref[b, :, _D:2 * _D] = e1[:_HIST]
        fut_ref[b] = df
        fut_ref[b, :, 0:_D] = e1[_HIST:]


def _dense_outputs(inputs, stage, m, bias, mf, biasf):
    return pl.pallas_call(
        _tc_body,
        grid=(_B // _BB,),
        in_specs=[
            pl.BlockSpec((_BB, _W, 8), lambda i: (i, 0, 0)),
            pl.BlockSpec((_BB, _W, _D), lambda i: (i, 0, 0)),
            pl.BlockSpec((8, _HC * _D), lambda i: (0, 0)),
            pl.BlockSpec((1, _HC * _D), lambda i: (0, 0)),
            pl.BlockSpec((8, _FC * _D), lambda i: (0, 0)),
            pl.BlockSpec((1, _FC * _D), lambda i: (0, 0)),
        ],
        out_specs=[
            pl.BlockSpec((_BB, _HIST, _HC * _D), lambda i: (i, 0, 0)),
            pl.BlockSpec((_BB, _FUT, _FC * _D), lambda i: (i, 0, 0)),
        ],
        out_shape=[
            jax.ShapeDtypeStruct((_B, _HIST, _HC * _D), jnp.float32),
            jax.ShapeDtypeStruct((_B, _FUT, _FC * _D), jnp.float32),
        ],
    )(inputs, stage, m, bias, mf, biasf)


def _static_gather(idx0, E0):
    bpw = _B // _NW
    mesh = plsc.VectorSubcoreMesh(core_axis_name="c", subcore_axis_name="s")

    @functools.partial(
        pl.kernel,
        mesh=mesh,
        out_type=jax.ShapeDtypeStruct((_B, _D), jnp.float32),
        compiler_params=pltpu.CompilerParams(use_tc_tiling_on_sc=False),
        scratch_types=[
            pltpu.VMEM((bpw,), jnp.int32),
            pltpu.VMEM((bpw, _D), jnp.float32),
            pltpu.SemaphoreType.DMA,
        ],
    )
    def k(idx_hbm, table_hbm, out_hbm, idx_v, rows_v, sem):
        wid = jax.lax.axis_index("s") * _NC + jax.lax.axis_index("c")
        base = wid * bpw
        pltpu.sync_copy(idx_hbm.at[pl.ds(base, bpw)], idx_v)
        pltpu.async_copy(table_hbm.at[idx_v], rows_v, sem).wait()
        pltpu.sync_copy(rows_v, out_hbm.at[pl.ds(base, bpw)])

    return k(idx0, E0)


def _e1_stage(idx1, E1):
    n = _B * _W
    npw = n // _NW
    nch = npw // _CHE
    mesh = plsc.VectorSubcoreMesh(core_axis_name="c", subcore_axis_name="s")

    nb = 4

    @functools.partial(
        pl.kernel,
        mesh=mesh,
        out_type=jax.ShapeDtypeStruct((n, _D), jnp.float32),
        compiler_params=pltpu.CompilerParams(use_tc_tiling_on_sc=False),
        scratch_types=[
            pltpu.VMEM((npw,), jnp.int32),
            pltpu.VMEM((nb, _CHE, _D), jnp.float32),
            [pltpu.SemaphoreType.DMA] * nb,
            [pltpu.SemaphoreType.DMA] * nb,
        ],
    )
    def k(idx_hbm, table_hbm, out_hbm, idx_v, rows_v, gsems, wsems):
        wid = jax.lax.axis_index("s") * _NC + jax.lax.axis_index("c")
        base = wid * npw
        pltpu.sync_copy(idx_hbm.at[pl.ds(base, npw)], idx_v)

        def gather(ch, sl):
            return pltpu.async_copy(
                table_hbm.at[idx_v.at[pl.ds(ch * _CHE, _CHE)]], rows_v.at[sl],
                gsems[sl])

        g = [None] * nb
        pend = [None] * nb
        for pre in range(min(2, nch)):
            g[pre] = gather(pre, pre)
        for ch in range(nch):
            sl = ch % nb
            g[sl].wait()
            pend[sl] = pltpu.async_copy(
                rows_v.at[sl], out_hbm.at[pl.ds(base + ch * _CHE, _CHE)], wsems[sl])
            nxt = ch + 2
            if nxt < nch:
                psl = nxt % nb
                if pend[psl] is not None:
                    pend[psl].wait()
                    pend[psl] = None
                g[psl] = gather(nxt, psl)
        for sl in range(nb):
            if pend[sl] is not None:
                pend[sl].wait()

    return k(idx1, E1)


def kernel(inputs, E0, E1, W2, b2, W3, b3, W4, b4, W5, b5, W6, b6, W7, b7):
    ws = {2: (W2, b2), 3: (W3, b3), 4: (W4, b4), 5: (W5, b5), 6: (W6, b6), 7: (W7, b7)}

    def proj(chans):
        mcols, bcols = [], []
        for v in chans:
            if v is None:
                mcols.append(jnp.zeros((8, _D), jnp.float32))
                bcols.append(jnp.zeros((_D,), jnp.float32))
            else:
                wv, bv = ws[v]
                mcols.append(jnp.zeros((8, _D), jnp.float32).at[v].set(wv[0]))
                bcols.append(bv)
        return jnp.concatenate(mcols, axis=1), jnp.concatenate(bcols)[None, :]

    m, bias = proj(_HIST_CH)
    mf, biasf = proj(_FUT_CH)

    idx1 = inputs[:, :, 1].astype(jnp.int32).reshape(-1)
    stage = _e1_stage(idx1, E1).reshape(_B, _W, _D)
    hist, fut = _dense_outputs(inputs, stage, m, bias, mf, biasf)

    idx0 = inputs[:, 0, 0].astype(jnp.int32)
    static = _static_gather(idx0, E0)

    return (
        static.reshape(_B, 1, _D),
        hist.reshape(_B, _HIST, _HC, _D),
        fut.reshape(_B, _FUT, _FC, _D),
    )
